# Initial kernel scaffold; baseline (speedup 1.0000x reference)
import jax
import jax.numpy as jnp
from jax import lax
from jax.experimental import pallas as pl
from jax.experimental.pallas import tpu as pltpu
from jax.experimental.pallas import tpu_sc as plsc

N = 10000
E = 640000
G = 64
IN = 2
HID = 32
HEADS = 4
EMB = 32
NC = 2

ET = E + N
NCORE = 2
NSUB = 16
NW = NCORE * NSUB
EPT = 20320
EPAD = NW * EPT

CH1 = 2032
NCH1 = EPT // CH1
CH2 = 4064
NCH2 = EPT // CH2
EQ = EPAD // 4
CH3 = 4064
NCH3 = EQ // CH3

_MESH = plsc.VectorSubcoreMesh(core_axis_name="c", subcore_axis_name="s")


def _wid():
    return lax.axis_index("c") * NSUB + lax.axis_index("s")


def _zero_ref(ref, n):
    z = jnp.zeros((16,), jnp.float32)

    def body(k, _):
        ref[pl.ds(k * 16, 16)] = z
        return 0

    lax.fori_loop(0, n // 16, body, 0)


def _dot16(a_ref, a_off, b_ref, b_off):
    return jnp.sum(a_ref[pl.ds(a_off, 16)] * b_ref[pl.ds(b_off, 16)])


def _l1a_body(x_hbm, w1_hbm, a1s_hbm, a1d_hbm, src_hbm, dst_hbm,
              p_out, den_out,
              x_t, w1_t, a1s_t, a1d_t, src_b, dst_b, p_b, den_acc):
    wid = _wid()
    pltpu.sync_copy(x_hbm, x_t)
    pltpu.sync_copy(w1_hbm, w1_t)
    pltpu.sync_copy(a1s_hbm, a1s_t)
    pltpu.sync_copy(a1d_hbm, a1d_t)
    _zero_ref(den_acc, HEADS * N)

    cs = []
    cd = []
    for i in range(IN):
        for h in range(HEADS):
            o = i * 128 + h * 32
            cs.append(_dot16(w1_t, o, a1s_t, h * 32)
                      + _dot16(w1_t, o + 16, a1s_t, h * 32 + 16))
            cd.append(_dot16(w1_t, o, a1d_t, h * 32)
                      + _dot16(w1_t, o + 16, a1d_t, h * 32 + 16))

    def chunk(c, _):
        base = wid * EPT + c * CH1
        pltpu.sync_copy(src_hbm.at[pl.ds(base, CH1)], src_b)
        pltpu.sync_copy(dst_hbm.at[pl.ds(base, CH1)], dst_b)

        def step(i, _):
            s16 = src_b[pl.ds(i * 16, 16)]
            d16 = dst_b[pl.ds(i * 16, 16)]
            s2 = s16 * 2
            d2 = d16 * 2
            x0s = plsc.load_gather(x_t, [s2])
            x1s = plsc.load_gather(x_t, [s2 + 1])
            x0d = plsc.load_gather(x_t, [d2])
            x1d = plsc.load_gather(x_t, [d2 + 1])
            eid = base + i * 16 + lax.iota(jnp.int32, 16)
            fm = jnp.where(eid < ET, 1.0, 0.0).astype(jnp.float32)
            for h in range(HEADS):
                e = (x0s * cs[h] + x1s * cs[HEADS + h]
                     + x0d * cd[h] + x1d * cd[HEADS + h])
                e = jnp.where(e > 0, e, 0.2 * e)
                p = jnp.exp(e) * fm
                plsc.addupdate_scatter(den_acc, [d16 + h * N], p)
                p_b[pl.ds(h * CH1 + i * 16, 16)] = p
            return 0

        lax.fori_loop(0, CH1 // 16, step, 0)
        pltpu.sync_copy(p_b, p_out.at[pl.ds((wid * NCH1 + c) * HEADS * CH1,
                                            HEADS * CH1)])
        return 0

    lax.fori_loop(0, NCH1, chunk, 0)
    pltpu.sync_copy(den_acc, den_out.at[pl.ds(wid * HEADS * N, HEADS * N)])


_l1a = pl.kernel(
    _l1a_body,
    out_type=(jax.ShapeDtypeStruct((EPAD * HEADS,), jnp.float32),
              jax.ShapeDtypeStruct((NW * HEADS * N,), jnp.float32)),
    mesh=_MESH,
    scratch_types=[
        pltpu.VMEM((N * IN,), jnp.float32),
        pltpu.VMEM((IN * HEADS * HID,), jnp.float32),
        pltpu.VMEM((HEADS * HID,), jnp.float32),
        pltpu.VMEM((HEADS * HID,), jnp.float32),
        pltpu.VMEM((CH1,), jnp.int32),
        pltpu.VMEM((CH1,), jnp.int32),
        pltpu.VMEM((HEADS * CH1,), jnp.float32),
        pltpu.VMEM((HEADS * N,), jnp.float32),
    ],
)


def _l1b_body(x_hbm, src_hbm, dst_hbm, p_hbm,
              num_out,
              x_t, src_b, dst_b, p_b, num_acc):
    wid = _wid()
    pltpu.sync_copy(x_hbm, x_t)
    _zero_ref(num_acc, HEADS * IN * N)

    def chunk(c, _):
        base = wid * EPT + c * CH1
        pltpu.sync_copy(src_hbm.at[pl.ds(base, CH1)], src_b)
        pltpu.sync_copy(dst_hbm.at[pl.ds(base, CH1)], dst_b)
        pltpu.sync_copy(p_hbm.at[pl.ds((wid * NCH1 + c) * HEADS * CH1,
                                       HEADS * CH1)], p_b)

        def step(i, _):
            s16 = src_b[pl.ds(i * 16, 16)]
            d16 = dst_b[pl.ds(i * 16, 16)]
            s2 = s16 * 2
            x0s = plsc.load_gather(x_t, [s2])
            x1s = plsc.load_gather(x_t, [s2 + 1])
            for h in range(HEADS):
                p = p_b[pl.ds(h * CH1 + i * 16, 16)]
                plsc.addupdate_scatter(num_acc, [d16 + (h * IN) * N], p * x0s)
                plsc.addupdate_scatter(num_acc, [d16 + (h * IN + 1) * N],
                                       p * x1s)
            return 0

        lax.fori_loop(0, CH1 // 16, step, 0)
        return 0

    lax.fori_loop(0, NCH1, chunk, 0)
    pltpu.sync_copy(num_acc,
                    num_out.at[pl.ds(wid * HEADS * IN * N, HEADS * IN * N)])


_l1b = pl.kernel(
    _l1b_body,
    out_type=jax.ShapeDtypeStruct((NW * HEADS * IN * N,), jnp.float32),
    mesh=_MESH,
    scratch_types=[
        pltpu.VMEM((N * IN,), jnp.float32),
        pltpu.VMEM((CH1,), jnp.int32),
        pltpu.VMEM((CH1,), jnp.int32),
        pltpu.VMEM((HEADS * CH1,), jnp.float32),
        pltpu.VMEM((HEADS * IN * N,), jnp.float32),
    ],
)


def _l2a_body(as_hbm, ad_hbm, src_hbm, dst_hbm,
              p_out, den_out,
              as_t, ad_t, src_b, dst_b, p_b, den_acc):
    wid = _wid()
    pltpu.sync_copy(as_hbm, as_t)
    pltpu.sync_copy(ad_hbm, ad_t)
    _zero_ref(den_acc, N)

    def chunk(c, _):
        base = wid * EPT + c * CH2
        pltpu.sync_copy(src_hbm.at[pl.ds(base, CH2)], src_b)
        pltpu.sync_copy(dst_hbm.at[pl.ds(base, CH2)], dst_b)

        def step(i, _):
            s16 = src_b[pl.ds(i * 16, 16)]
            d16 = dst_b[pl.ds(i * 16, 16)]
            av = plsc.load_gather(as_t, [s16])
            bv = plsc.load_gather(ad_t, [d16])
            e = av + bv
            e = jnp.where(e > 0, e, 0.2 * e)
            eid = base + i * 16 + lax.iota(jnp.int32, 16)
            fm = jnp.where(eid < ET, 1.0, 0.0).astype(jnp.float32)
            p = jnp.exp(e) * fm
            plsc.addupdate_scatter(den_acc, [d16], p)
            p_b[pl.ds(i * 16, 16)] = p
            return 0

        lax.fori_loop(0, CH2 // 16, step, 0)
        pltpu.sync_copy(p_b, p_out.at[pl.ds(base, CH2)])
        return 0

    lax.fori_loop(0, NCH2, chunk, 0)
    pltpu.sync_copy(den_acc, den_out.at[pl.ds(wid * N, N)])


_l2a = pl.kernel(
    _l2a_body,
    out_type=(jax.ShapeDtypeStruct((EPAD,), jnp.float32),
              jax.ShapeDtypeStruct((NW * N,), jnp.float32)),
    mesh=_MESH,
    scratch_types=[
        pltpu.VMEM((N,), jnp.float32),
        pltpu.VMEM((N,), jnp.float32),
        pltpu.VMEM((CH2,), jnp.int32),
        pltpu.VMEM((CH2,), jnp.int32),
        pltpu.VMEM((CH2,), jnp.float32),
        pltpu.VMEM((N,), jnp.float32),
    ],
)


def _l2b_body(h2_hbm, src_hbm, dst_hbm, p_hbm,
              num_out,
              h2_t, src_b, dst_b, p_b, num_acc):
    wid = _wid()
    g = wid % 8
    q = wid // 8
    pltpu.sync_copy(h2_hbm.at[pl.ds(g * 4 * N, 4 * N)], h2_t)
    _zero_ref(num_acc, 4 * N)

    def chunk(c, _):
        base = q * EQ + c * CH3
        pltpu.sync_copy(src_hbm.at[pl.ds(base, CH3)], src_b)
        pltpu.sync_copy(dst_hbm.at[pl.ds(base, CH3)], dst_b)
        pltpu.sync_copy(p_hbm.at[pl.ds(base, CH3)], p_b)

        def step(i, _):
            s16 = src_b[pl.ds(i * 16, 16)]
            d16 = dst_b[pl.ds(i * 16, 16)]
            p = p_b[pl.ds(i * 16, 16)]
            for j in range(4):
                hv = plsc.load_gather(h2_t, [s16 + j * N])
                plsc.addupdate_scatter(num_acc, [d16 + j * N], p * hv)
            return 0

        lax.fori_loop(0, CH3 // 16, step, 0)
        return 0

    lax.fori_loop(0, NCH3, chunk, 0)
    pltpu.sync_copy(num_acc, num_out.at[pl.ds(wid * 4 * N, 4 * N)])


_l2b = pl.kernel(
    _l2b_body,
    out_type=jax.ShapeDtypeStruct((NW * 4 * N,), jnp.float32),
    mesh=_MESH,
    scratch_types=[
        pltpu.VMEM((4 * N,), jnp.float32),
        pltpu.VMEM((CH3,), jnp.int32),
        pltpu.VMEM((CH3,), jnp.int32),
        pltpu.VMEM((CH3,), jnp.float32),
        pltpu.VMEM((4 * N,), jnp.float32),
    ],
)


def _tc_mid_body(den_p, num_p, w1t, b1, g1, be1, w2, as2w, ad2w,
                 h2t_o, as2_o, ad2_o):
    den = jnp.sum(den_p[...], axis=0)
    num = jnp.sum(num_p[...], axis=0)
    den_r = jnp.broadcast_to(den.reshape(HEADS, 1, N),
                             (HEADS, IN, N)).reshape(HEADS * IN, N)
    qn = num / (den_r + 1e-16)
    wt = jnp.concatenate([w1t[...]] * HEADS, axis=1)
    r = lax.broadcasted_iota(jnp.int32, (HEADS * HID, HEADS * IN), 0)
    c = lax.broadcasted_iota(jnp.int32, (HEADS * HID, HEADS * IN), 1)
    bd = jnp.where(r // HID == c // IN, wt, 0.0)
    h = jnp.dot(bd, qn, preferred_element_type=jnp.float32) + b1[...]
    mu = jnp.mean(h, axis=1, keepdims=True)
    var = jnp.mean((h - mu) ** 2, axis=1, keepdims=True)
    h = (h - mu) / jnp.sqrt(var + 1e-5) * g1[...] + be1[...]
    h = jnp.where(h > 0, h, jnp.exp(jnp.minimum(h, 0.0)) - 1.0)
    h2 = lax.dot_general(w2[...], h, (((0,), (0,)), ((), ())),
                         preferred_element_type=jnp.float32)
    h2t_o[...] = h2
    as2_o[...] = jnp.dot(as2w[...], h2, preferred_element_type=jnp.float32)
    ad2_o[...] = jnp.dot(ad2w[...], h2, preferred_element_type=jnp.float32)


def _tc_mid(den_p, num_p, w1t, b1, g1, be1, w2, as2w, ad2w):
    return pl.pallas_call(
        _tc_mid_body,
        out_shape=(jax.ShapeDtypeStruct((EMB, N), jnp.float32),
                   jax.ShapeDtypeStruct((1, N), jnp.float32),
                   jax.ShapeDtypeStruct((1, N), jnp.float32)),
    )(den_p, num_p, w1t, b1, g1, be1, w2, as2w, ad2w)


def _tc_out_body(den_p, num_p, batch, b2, g2, be2, wc, bc, out):
    den = jnp.sum(den_p[...], axis=0).reshape(1, N)
    num = jnp.sum(num_p[...].reshape(4, 8, 4, N), axis=0).reshape(EMB, N)
    h = num / (den + 1e-16) + b2[...]
    mu = jnp.mean(h, axis=1, keepdims=True)
    var = jnp.mean((h - mu) ** 2, axis=1, keepdims=True)
    h = (h - mu) / jnp.sqrt(var + 1e-5) * g2[...] + be2[...]
    h = jnp.where(h > 0, h, jnp.exp(jnp.minimum(h, 0.0)) - 1.0)
    seg = (batch[...] == lax.broadcasted_iota(jnp.int32, (N, G), 1))
    seg = seg.astype(jnp.float32)
    s = jnp.dot(h, seg, preferred_element_type=jnp.float32)
    cnt = jnp.sum(seg, axis=0, keepdims=True)
    emb = s / jnp.maximum(cnt, 1.0)
    out[...] = lax.dot_general(emb, wc[...], (((0,), (0,)), ((), ())),
                               preferred_element_type=jnp.float32) + bc[...]


def _tc_out(den_p, num_p, batch, b2, g2, be2, wc, bc):
    return pl.pallas_call(
        _tc_out_body,
        out_shape=jax.ShapeDtypeStruct((G, NC), jnp.float32),
    )(den_p, num_p, batch, b2, g2, be2, wc, bc)


def kernel(x, edge_index, batch, W1, a_src1, a_dst1, b1, g1, be1,
           W2, a_src2, a_dst2, b2, g2, be2, Wc, bc):
    loop = jnp.arange(N, dtype=jnp.int32)
    padi = jnp.zeros((EPAD - ET,), jnp.int32)
    src = jnp.concatenate([edge_index[0], loop, padi])
    dst = jnp.concatenate([edge_index[1], loop, padi])

    p1, den1p = _l1a(x.reshape(-1), W1.reshape(-1), a_src1.reshape(-1),
                     a_dst1.reshape(-1), src, dst)
    num1p = _l1b(x.reshape(-1), src, dst, p1)
    h2t, as2, ad2 = _tc_mid(den1p.reshape(NW, HEADS, N),
                            num1p.reshape(NW, HEADS * IN, N),
                            W1.T, b1.reshape(-1, 1), g1.reshape(-1, 1),
                            be1.reshape(-1, 1), W2, a_src2, a_dst2)
    p2, den2p = _l2a(as2.reshape(-1), ad2.reshape(-1), src, dst)
    num2p = _l2b(h2t.reshape(-1), src, dst, p2)
    out = _tc_out(den2p.reshape(NW, N), num2p.reshape(NW, 4, N),
                  batch.reshape(-1, 1), b2.reshape(-1, 1),
                  g2.reshape(-1, 1), be2.reshape(-1, 1), Wc,
                  bc.reshape(1, -1))
    return out


# speedup vs baseline: 117.0223x; 117.0223x over previous
import jax
import jax.numpy as jnp
from jax import lax
from jax.experimental import pallas as pl
from jax.experimental.pallas import tpu as pltpu
from jax.experimental.pallas import tpu_sc as plsc

N = 10000
E = 640000
G = 64
IN = 2
HID = 32
HEADS = 4
EMB = 32
NC = 2

ET = E + N
NCORE = 2
NSUB = 16
NW = NCORE * NSUB
EPT = 20480
EPAD = NW * EPT

CH1 = 1024
NCH1 = EPT // CH1
CH2 = 4096
NCH2 = EPT // CH2
EQ = EPAD // 4
CH3 = 4096
NCH3 = EQ // CH3

_MESH = plsc.VectorSubcoreMesh(core_axis_name="c", subcore_axis_name="s")
_SC_PARAMS = pltpu.CompilerParams(needs_layout_passes=False)


def _wid():
    return lax.axis_index("c") * NSUB + lax.axis_index("s")


def _zero_ref(ref, n):
    z = jnp.zeros((16,), jnp.float32)

    def body(k, _):
        ref[pl.ds(k * 16, 16)] = z
        return 0

    lax.fori_loop(0, n // 16, body, 0)


def _l1a_body(as_hbm, ad_hbm, src_hbm, dst_hbm,
              p_out, den_out,
              as_t, ad_t, src_b, dst_b, p_b, den_acc):
    wid = _wid()
    pltpu.sync_copy(as_hbm, as_t)
    pltpu.sync_copy(ad_hbm, ad_t)
    _zero_ref(den_acc, HEADS * N)

    def chunk(c, _):
        base = wid * EPT + c * CH1
        pltpu.sync_copy(src_hbm.at[pl.ds(base, CH1)], src_b)
        pltpu.sync_copy(dst_hbm.at[pl.ds(base, CH1)], dst_b)

        def step(i, _):
            s16 = src_b[pl.ds(i * 16, 16)]
            d16 = dst_b[pl.ds(i * 16, 16)]
            s4 = s16 * 4
            d4 = d16 * 4
            eid = base + i * 16 + lax.iota(jnp.int32, 16)
            fm = jnp.where(eid < ET, 1.0, 0.0).astype(jnp.float32)
            for h in range(HEADS):
                av = plsc.load_gather(as_t, [s4 + h])
                bv = plsc.load_gather(ad_t, [d4 + h])
                e = av + bv
                e = jnp.where(e > 0, e, 0.2 * e)
                p = jnp.exp(e) * fm
                plsc.addupdate_scatter(den_acc, [d16 + h * N], p)
                p_b[pl.ds(h * CH1 + i * 16, 16)] = p
            return 0

        lax.fori_loop(0, CH1 // 16, step, 0)
        pltpu.sync_copy(p_b, p_out.at[pl.ds((wid * NCH1 + c) * HEADS * CH1,
                                            HEADS * CH1)])
        return 0

    lax.fori_loop(0, NCH1, chunk, 0)
    pltpu.sync_copy(den_acc, den_out.at[pl.ds(wid * HEADS * N, HEADS * N)])


_l1a = pl.kernel(
    _l1a_body,
    out_type=(jax.ShapeDtypeStruct((EPAD * HEADS,), jnp.float32),
              jax.ShapeDtypeStruct((NW * HEADS * N,), jnp.float32)),
    mesh=_MESH,
    compiler_params=_SC_PARAMS,
    scratch_types=[
        pltpu.VMEM((HEADS * N,), jnp.float32),
        pltpu.VMEM((HEADS * N,), jnp.float32),
        pltpu.VMEM((CH1,), jnp.int32),
        pltpu.VMEM((CH1,), jnp.int32),
        pltpu.VMEM((HEADS * CH1,), jnp.float32),
        pltpu.VMEM((HEADS * N,), jnp.float32),
    ],
)


def _tc_pre_body(x, w1, a1s, a1d, as_o, ad_o):
    w1r = w1[...].reshape(IN, HEADS, HID)
    cs = jnp.sum(w1r * a1s[...][None], axis=-1)
    cd = jnp.sum(w1r * a1d[...][None], axis=-1)
    as_o[...] = jnp.dot(x[...], cs, preferred_element_type=jnp.float32)
    ad_o[...] = jnp.dot(x[...], cd, preferred_element_type=jnp.float32)


def _tc_pre(x, w1, a1s, a1d):
    return pl.pallas_call(
        _tc_pre_body,
        out_shape=(jax.ShapeDtypeStruct((N, HEADS), jnp.float32),
                   jax.ShapeDtypeStruct((N, HEADS), jnp.float32)),
    )(x, w1, a1s, a1d)


def _l1b_body(x_hbm, src_hbm, dst_hbm, p_hbm,
              num_out,
              x_t, src_b, dst_b, p_b, num_acc):
    wid = _wid()
    pltpu.sync_copy(x_hbm, x_t)
    _zero_ref(num_acc, HEADS * IN * N)

    def chunk(c, _):
        base = wid * EPT + c * CH1
        pltpu.sync_copy(src_hbm.at[pl.ds(base, CH1)], src_b)
        pltpu.sync_copy(dst_hbm.at[pl.ds(base, CH1)], dst_b)
        pltpu.sync_copy(p_hbm.at[pl.ds((wid * NCH1 + c) * HEADS * CH1,
                                       HEADS * CH1)], p_b)

        def step(i, _):
            s16 = src_b[pl.ds(i * 16, 16)]
            d16 = dst_b[pl.ds(i * 16, 16)]
            s2 = s16 * 2
            x0s = plsc.load_gather(x_t, [s2])
            x1s = plsc.load_gather(x_t, [s2 + 1])
            for h in range(HEADS):
                p = p_b[pl.ds(h * CH1 + i * 16, 16)]
                plsc.addupdate_scatter(num_acc, [d16 + (h * IN) * N], p * x0s)
                plsc.addupdate_scatter(num_acc, [d16 + (h * IN + 1) * N],
                                       p * x1s)
            return 0

        lax.fori_loop(0, CH1 // 16, step, 0)
        return 0

    lax.fori_loop(0, NCH1, chunk, 0)
    pltpu.sync_copy(num_acc,
                    num_out.at[pl.ds(wid * HEADS * IN * N, HEADS * IN * N)])


_l1b = pl.kernel(
    _l1b_body,
    out_type=jax.ShapeDtypeStruct((NW * HEADS * IN * N,), jnp.float32),
    mesh=_MESH,
    compiler_params=_SC_PARAMS,
    scratch_types=[
        pltpu.VMEM((N * IN,), jnp.float32),
        pltpu.VMEM((CH1,), jnp.int32),
        pltpu.VMEM((CH1,), jnp.int32),
        pltpu.VMEM((HEADS * CH1,), jnp.float32),
        pltpu.VMEM((HEADS * IN * N,), jnp.float32),
    ],
)


def _l2a_body(as_hbm, ad_hbm, src_hbm, dst_hbm,
              p_out, den_out,
              as_t, ad_t, src_b, dst_b, p_b, den_acc):
    wid = _wid()
    pltpu.sync_copy(as_hbm, as_t)
    pltpu.sync_copy(ad_hbm, ad_t)
    _zero_ref(den_acc, N)

    def chunk(c, _):
        base = wid * EPT + c * CH2
        pltpu.sync_copy(src_hbm.at[pl.ds(base, CH2)], src_b)
        pltpu.sync_copy(dst_hbm.at[pl.ds(base, CH2)], dst_b)

        def step(i, _):
            s16 = src_b[pl.ds(i * 16, 16)]
            d16 = dst_b[pl.ds(i * 16, 16)]
            av = plsc.load_gather(as_t, [s16])
            bv = plsc.load_gather(ad_t, [d16])
            e = av + bv
            e = jnp.where(e > 0, e, 0.2 * e)
            eid = base + i * 16 + lax.iota(jnp.int32, 16)
            fm = jnp.where(eid < ET, 1.0, 0.0).astype(jnp.float32)
            p = jnp.exp(e) * fm
            plsc.addupdate_scatter(den_acc, [d16], p)
            p_b[pl.ds(i * 16, 16)] = p
            return 0

        lax.fori_loop(0, CH2 // 16, step, 0)
        pltpu.sync_copy(p_b, p_out.at[pl.ds(base, CH2)])
        return 0

    lax.fori_loop(0, NCH2, chunk, 0)
    pltpu.sync_copy(den_acc, den_out.at[pl.ds(wid * N, N)])


_l2a = pl.kernel(
    _l2a_body,
    out_type=(jax.ShapeDtypeStruct((EPAD,), jnp.float32),
              jax.ShapeDtypeStruct((NW * N,), jnp.float32)),
    mesh=_MESH,
    compiler_params=_SC_PARAMS,
    scratch_types=[
        pltpu.VMEM((N,), jnp.float32),
        pltpu.VMEM((N,), jnp.float32),
        pltpu.VMEM((CH2,), jnp.int32),
        pltpu.VMEM((CH2,), jnp.int32),
        pltpu.VMEM((CH2,), jnp.float32),
        pltpu.VMEM((N,), jnp.float32),
    ],
)


def _l2b_body(h2_hbm, src_hbm, dst_hbm, p_hbm,
              num_out,
              h2_t, src_b, dst_b, p_b, num_acc):
    wid = _wid()
    g = wid % 8
    q = wid // 8
    pltpu.sync_copy(h2_hbm.at[pl.ds(g * 4 * N, 4 * N)], h2_t)
    _zero_ref(num_acc, 4 * N)

    def chunk(c, _):
        base = q * EQ + c * CH3
        pltpu.sync_copy(src_hbm.at[pl.ds(base, CH3)], src_b)
        pltpu.sync_copy(dst_hbm.at[pl.ds(base, CH3)], dst_b)
        pltpu.sync_copy(p_hbm.at[pl.ds(base, CH3)], p_b)

        def step(i, _):
            s16 = src_b[pl.ds(i * 16, 16)]
            d16 = dst_b[pl.ds(i * 16, 16)]
            p = p_b[pl.ds(i * 16, 16)]
            for j in range(4):
                hv = plsc.load_gather(h2_t, [s16 + j * N])
                plsc.addupdate_scatter(num_acc, [d16 + j * N], p * hv)
            return 0

        lax.fori_loop(0, CH3 // 16, step, 0)
        return 0

    lax.fori_loop(0, NCH3, chunk, 0)
    pltpu.sync_copy(num_acc, num_out.at[pl.ds(wid * 4 * N, 4 * N)])


_l2b = pl.kernel(
    _l2b_body,
    out_type=jax.ShapeDtypeStruct((NW * 4 * N,), jnp.float32),
    mesh=_MESH,
    compiler_params=_SC_PARAMS,
    scratch_types=[
        pltpu.VMEM((4 * N,), jnp.float32),
        pltpu.VMEM((CH3,), jnp.int32),
        pltpu.VMEM((CH3,), jnp.int32),
        pltpu.VMEM((CH3,), jnp.float32),
        pltpu.VMEM((4 * N,), jnp.float32),
    ],
)


def _tc_mid_body(den_p, num_p, w1t, b1, g1, be1, w2, as2w, ad2w,
                 h2t_o, as2_o, ad2_o):
    den = jnp.sum(den_p[...], axis=0)
    num = jnp.sum(num_p[...], axis=0)
    den_r = jnp.broadcast_to(den.reshape(HEADS, 1, N),
                             (HEADS, IN, N)).reshape(HEADS * IN, N)
    qn = num / (den_r + 1e-16)
    wt = jnp.concatenate([w1t[...]] * HEADS, axis=1)
    r = lax.broadcasted_iota(jnp.int32, (HEADS * HID, HEADS * IN), 0)
    c = lax.broadcasted_iota(jnp.int32, (HEADS * HID, HEADS * IN), 1)
    bd = jnp.where(r // HID == c // IN, wt, 0.0)
    h = jnp.dot(bd, qn, preferred_element_type=jnp.float32) + b1[...]
    mu = jnp.mean(h, axis=1, keepdims=True)
    var = jnp.mean((h - mu) ** 2, axis=1, keepdims=True)
    h = (h - mu) / jnp.sqrt(var + 1e-5) * g1[...] + be1[...]
    h = jnp.where(h > 0, h, jnp.exp(jnp.minimum(h, 0.0)) - 1.0)
    h2 = lax.dot_general(w2[...], h, (((0,), (0,)), ((), ())),
                         preferred_element_type=jnp.float32)
    h2t_o[...] = h2
    as2_o[...] = jnp.dot(as2w[...], h2, preferred_element_type=jnp.float32)
    ad2_o[...] = jnp.dot(ad2w[...], h2, preferred_element_type=jnp.float32)


def _tc_mid(den_p, num_p, w1t, b1, g1, be1, w2, as2w, ad2w):
    return pl.pallas_call(
        _tc_mid_body,
        out_shape=(jax.ShapeDtypeStruct((EMB, N), jnp.float32),
                   jax.ShapeDtypeStruct((1, N), jnp.float32),
                   jax.ShapeDtypeStruct((1, N), jnp.float32)),
    )(den_p, num_p, w1t, b1, g1, be1, w2, as2w, ad2w)


def _tc_out_body(den_p, num_p, batch, b2, g2, be2, wc, bc, out):
    den = jnp.sum(den_p[...], axis=0).reshape(1, N)
    num = jnp.sum(num_p[...].reshape(4, 8, 4, N), axis=0).reshape(EMB, N)
    h = num / (den + 1e-16) + b2[...]
    mu = jnp.mean(h, axis=1, keepdims=True)
    var = jnp.mean((h - mu) ** 2, axis=1, keepdims=True)
    h = (h - mu) / jnp.sqrt(var + 1e-5) * g2[...] + be2[...]
    h = jnp.where(h > 0, h, jnp.exp(jnp.minimum(h, 0.0)) - 1.0)
    seg = (batch[...] == lax.broadcasted_iota(jnp.int32, (N, G), 1))
    seg = seg.astype(jnp.float32)
    s = jnp.dot(h, seg, preferred_element_type=jnp.float32)
    cnt = jnp.sum(seg, axis=0, keepdims=True)
    emb = s / jnp.maximum(cnt, 1.0)
    out[...] = lax.dot_general(emb, wc[...], (((0,), (0,)), ((), ())),
                               preferred_element_type=jnp.float32) + bc[...]


def _tc_out(den_p, num_p, batch, b2, g2, be2, wc, bc):
    return pl.pallas_call(
        _tc_out_body,
        out_shape=jax.ShapeDtypeStruct((G, NC), jnp.float32),
    )(den_p, num_p, batch, b2, g2, be2, wc, bc)


def kernel(x, edge_index, batch, W1, a_src1, a_dst1, b1, g1, be1,
           W2, a_src2, a_dst2, b2, g2, be2, Wc, bc):
    loop = jnp.arange(N, dtype=jnp.int32)
    padi = jnp.zeros((EPAD - ET,), jnp.int32)
    src = jnp.concatenate([edge_index[0], loop, padi])
    dst = jnp.concatenate([edge_index[1], loop, padi])

    as1, ad1 = _tc_pre(x, W1, a_src1, a_dst1)
    p1, den1p = _l1a(as1.reshape(-1), ad1.reshape(-1), src, dst)
    num1p = _l1b(x.reshape(-1), src, dst, p1)
    h2t, as2, ad2 = _tc_mid(den1p.reshape(NW, HEADS, N),
                            num1p.reshape(NW, HEADS * IN, N),
                            W1.T, b1.reshape(-1, 1), g1.reshape(-1, 1),
                            be1.reshape(-1, 1), W2, a_src2, a_dst2)
    p2, den2p = _l2a(as2.reshape(-1), ad2.reshape(-1), src, dst)
    num2p = _l2b(h2t.reshape(-1), src, dst, p2)
    out = _tc_out(den2p.reshape(NW, N), num2p.reshape(NW, 4, N),
                  batch.reshape(-1, 1), b2.reshape(-1, 1),
                  g2.reshape(-1, 1), be2.reshape(-1, 1), Wc,
                  bc.reshape(1, -1))
    return out
